# baseline (speedup 1.0000x reference)
import functools

import jax
import jax.numpy as jnp
from jax import lax
from jax.experimental import pallas as pl
from jax.experimental.pallas import tpu as pltpu
from jax.experimental.pallas import tpu_sc as plsc

_info = plsc.get_sparse_core_info()
_NC, _NS = _info.num_cores, _info.num_subcores
_NW = _NC * _NS


_NCHUNKS = 4


def _make_gather(num_nodes, dim, batch_size):
  assert batch_size % (8 * _NW) == 0
  b_per_w = batch_size // _NW
  assert b_per_w % _NCHUNKS == 0
  chunk = b_per_w // _NCHUNKS
  mesh = plsc.VectorSubcoreMesh(core_axis_name="c", subcore_axis_name="s")

  @functools.partial(
      pl.kernel,
      mesh=mesh,
      out_type=jax.ShapeDtypeStruct((batch_size, dim), jnp.float32),
      scratch_types=[
          pltpu.VMEM((b_per_w,), jnp.int32),
          pltpu.VMEM((b_per_w, dim), jnp.float32),
      ]
      + [pltpu.SemaphoreType.DMA] * (2 * _NCHUNKS),
  )
  def gather_kernel(table_hbm, idx_hbm, out_hbm, idx_v, rows_v, *sems):
    gsems, osems = sems[:_NCHUNKS], sems[_NCHUNKS:]
    wid = lax.axis_index("s") * _NC + lax.axis_index("c")
    base = wid * b_per_w
    pltpu.sync_copy(idx_hbm.at[pl.ds(base, b_per_w)], idx_v)
    gathers = [
        pltpu.async_copy(
            table_hbm.at[idx_v.at[pl.ds(c * chunk, chunk)]],
            rows_v.at[pl.ds(c * chunk, chunk)],
            gsems[c],
        )
        for c in range(_NCHUNKS)
    ]
    writebacks = []
    for c in range(_NCHUNKS):
      gathers[c].wait()
      writebacks.append(
          pltpu.async_copy(
              rows_v.at[pl.ds(c * chunk, chunk)],
              out_hbm.at[pl.ds(base + c * chunk, chunk)],
              osems[c],
          )
      )
    for wb in writebacks:
      wb.wait()

  return gather_kernel


@jax.jit
def kernel(batch, embedding):
  num_nodes, dim = embedding.shape
  (batch_size,) = batch.shape
  return _make_gather(num_nodes, dim, batch_size)(embedding, batch)


# speedup vs baseline: 1.0163x; 1.0163x over previous
import functools

import jax
import jax.numpy as jnp
from jax import lax
from jax.experimental import pallas as pl
from jax.experimental.pallas import tpu as pltpu
from jax.experimental.pallas import tpu_sc as plsc

_info = plsc.get_sparse_core_info()
_NC, _NS = _info.num_cores, _info.num_subcores
_NW = _NC * _NS


def _make_gather(num_nodes, dim, batch_size):
  assert batch_size % (8 * _NW) == 0
  b_per_w = batch_size // _NW
  mesh = plsc.VectorSubcoreMesh(core_axis_name="c", subcore_axis_name="s")

  @functools.partial(
      pl.kernel,
      mesh=mesh,
      out_type=jax.ShapeDtypeStruct((batch_size, dim), jnp.float32),
      scratch_types=[
          pltpu.VMEM((b_per_w,), jnp.int32),
          pltpu.VMEM((b_per_w, dim), jnp.float32),
          pltpu.SemaphoreType.DMA,
      ],
  )
  def gather_kernel(table_hbm, idx_hbm, out_hbm, idx_v, rows_v, sem):
    wid = lax.axis_index("s") * _NC + lax.axis_index("c")
    base = wid * b_per_w
    pltpu.sync_copy(idx_hbm.at[pl.ds(base, b_per_w)], idx_v)
    pltpu.async_copy(table_hbm.at[idx_v], rows_v, sem).wait()
    pltpu.sync_copy(rows_v, out_hbm.at[pl.ds(base, b_per_w)])

  return gather_kernel


@jax.jit
def kernel(batch, embedding):
  num_nodes, dim = embedding.shape
  (batch_size,) = batch.shape
  return _make_gather(num_nodes, dim, batch_size)(embedding, batch)


# speedup vs baseline: 1.0207x; 1.0044x over previous
import functools

import jax
import jax.numpy as jnp
from jax import lax
from jax.experimental import pallas as pl
from jax.experimental.pallas import tpu as pltpu
from jax.experimental.pallas import tpu_sc as plsc

_info = plsc.get_sparse_core_info()
_NC, _NS = _info.num_cores, _info.num_subcores
_NW = _NC * _NS


def _make_gather(num_nodes, dim, batch_size):
  assert batch_size % (8 * _NW) == 0
  b_per_w = batch_size // _NW
  mesh = plsc.VectorSubcoreMesh(core_axis_name="c", subcore_axis_name="s")

  @functools.partial(
      pl.kernel,
      mesh=mesh,
      out_type=jax.ShapeDtypeStruct((batch_size, dim), jnp.float32),
      scratch_types=[
          pltpu.VMEM((b_per_w,), jnp.int32),
          pltpu.VMEM((b_per_w, dim), jnp.float32),
          pltpu.SemaphoreType.DMA,
      ],
  )
  def gather_kernel(table_hbm, idx_hbm, out_hbm, idx_v, rows_v, sem):
    wid = lax.axis_index("c") * _NS + lax.axis_index("s")
    base = wid * b_per_w
    pltpu.sync_copy(idx_hbm.at[pl.ds(base, b_per_w)], idx_v)
    pltpu.async_copy(table_hbm.at[idx_v], rows_v, sem).wait()
    pltpu.sync_copy(rows_v, out_hbm.at[pl.ds(base, b_per_w)])

  return gather_kernel


@jax.jit
def kernel(batch, embedding):
  num_nodes, dim = embedding.shape
  (batch_size,) = batch.shape
  return _make_gather(num_nodes, dim, batch_size)(embedding, batch)
